# baseline (speedup 1.0000x reference)
import jax
import jax.numpy as jnp
from jax import lax
from jax.experimental import pallas as pl
from jax.experimental.pallas import tpu as pltpu
from jax.experimental.pallas import tpu_sc as plsc

NCIR = 585
NDIS = 88
D = 128
E_CC = 18720
E_DD = 1408

CC_N = 640
DD_N = 128
NCORE = 2
NTILE = 16
CC_ROWS = CC_N // NCORE
DD_ROWS = DD_N // NCORE
F_CC = CC_ROWS * CC_N
F_DD = DD_ROWS * DD_N
CC_CH = 1280
CC_NB = CC_CH // 128
DD_CH = 128
DD_NT = E_DD // DD_CH
CC_SL = F_CC // NTILE
PLANE = CC_ROWS * 128
NKP = CC_N // 128
DD_SL = F_DD // NTILE
ZCH = 2560
CC_FULL = E_CC // CC_CH
CC_BLK = 18688
CC_T14 = CC_BLK - CC_FULL * CC_CH
CC_T32 = E_CC - CC_BLK

_f32 = jnp.float32
_i32 = jnp.int32
_i16 = jnp.int16


def _sc_body(cc_e2, cc_tl, dd_e2,
             out_cc, out_dd,
             acc_cc, acc_dd,
             ecc, etl, edd,
             sidxs, wbufs, zbuf, sem_z, sem_s, sem_e):
    c = lax.axis_index("c")
    s = lax.axis_index("s")

    @pl.when(s < CC_FULL)
    def _():
        pltpu.async_copy(cc_e2.at[:, pl.ds(s * CC_CH, CC_CH)], ecc, sem_e)

    @pl.when(s == CC_FULL)
    def _():
        pltpu.async_copy(cc_e2.at[:, pl.ds(CC_FULL * CC_CH, CC_T14)],
                         ecc.at[:, pl.ds(0, CC_T14)], sem_e)

    @pl.when(s == NTILE - 1)
    def _():
        pltpu.async_copy(cc_tl, etl, sem_e)

    @pl.when(s < DD_NT)
    def _():
        pltpu.async_copy(dd_e2.at[:, pl.ds(s * DD_CH, DD_CH)], edd, sem_e)

    zv = jnp.zeros((16,), _f32)

    def zloop(i, carry):
        base = i * 64
        zbuf[pl.ds(base, 16)] = zv
        zbuf[pl.ds(base + 16, 16)] = zv
        zbuf[pl.ds(base + 32, 16)] = zv
        zbuf[pl.ds(base + 48, 16)] = zv
        return carry

    lax.fori_loop(0, ZCH // 64, zloop, 0)
    zds = [pltpu.async_copy(zbuf, acc_cc.at[pl.ds(s * CC_SL + k * ZCH, ZCH)],
                            sem_z) for k in range(CC_SL // ZCH)]
    zds.append(pltpu.async_copy(zbuf.at[pl.ds(0, DD_SL)],
                                acc_dd.at[pl.ds(s * DD_SL, DD_SL)], sem_z))

    @pl.when(s < CC_FULL)
    def _():
        pltpu.make_async_copy(cc_e2.at[:, pl.ds(0, CC_CH)], ecc, sem_e).wait()

    @pl.when(s == CC_FULL)
    def _():
        pltpu.make_async_copy(cc_e2.at[:, pl.ds(0, CC_T14)],
                              ecc.at[:, pl.ds(0, CC_T14)], sem_e).wait()

    @pl.when(s == NTILE - 1)
    def _():
        pltpu.make_async_copy(cc_tl, etl, sem_e).wait()

    @pl.when(s < DD_NT)
    def _():
        pltpu.make_async_copy(dd_e2.at[:, pl.ds(0, DD_CH)], edd, sem_e).wait()

    lane = lax.iota(_i32, 16)
    row_lo = c * CC_ROWS

    def make_idx_cc(j, sr, wr):
        def body(i, carry):
            off = j * 128 + i * 16
            sv = ecc[0, pl.ds(off, 16)]
            dv = ecc[1, pl.ds(off, 16)]
            loc = s * CC_CH + off + lane
            valid = (loc < CC_BLK) & (sv >= row_lo) & (sv < row_lo + CC_ROWS)
            sr[pl.ds(i * 16, 16)] = jnp.where(
                valid,
                (dv >> 7) * PLANE + (sv - row_lo) * 128 + (dv & 127),
                s * CC_CH + off + lane)
            wr[pl.ds(i * 16, 16)] = jnp.where(valid, 1.0, 0.0)
            return carry
        lax.fori_loop(0, 8, body, 0)

    for j in range(CC_NB):
        make_idx_cc(j, sidxs[j], wbufs[j])

    @pl.when(s == NTILE - 1)
    def _():
        for i in range(CC_T32 // 16):
            sv = etl[pl.ds(i * 16, 16)]
            dv = etl[pl.ds(CC_T32 + i * 16, 16)]
            valid = (sv >= row_lo) & (sv < row_lo + CC_ROWS)
            sidxs[0][pl.ds(i * 16, 16)] = jnp.where(
                valid,
                (dv >> 7) * PLANE + (sv - row_lo) * 128 + (dv & 127),
                i * 16 + lane)
            wbufs[0][pl.ds(i * 16, 16)] = jnp.where(valid, 1.0, 0.0)

    dd_lo = c * DD_ROWS

    def dd_body(i, carry):
        off = i * 16
        sv = edd[0, pl.ds(off, 16)]
        dv = edd[1, pl.ds(off, 16)]
        gd = s * DD_CH + off + lane
        valid = (gd < E_DD) & (sv >= dd_lo) & (sv < dd_lo + DD_ROWS)
        sidxs[CC_NB][pl.ds(off, 16)] = jnp.where(
            valid, (sv - dd_lo) * DD_N + dv, s * DD_CH + off + lane)
        wbufs[CC_NB][pl.ds(off, 16)] = jnp.where(valid, 1.0, 0.0)
        return carry

    lax.fori_loop(0, 8, dd_body, 0)

    for z in zds:
        z.wait()

    plsc.subcore_barrier()

    sds = [pltpu.async_copy(wbufs[j], acc_cc.at[sidxs[j]], sem_s, add=True)
           for j in range(CC_NB)]
    sds.append(pltpu.async_copy(wbufs[CC_NB], acc_dd.at[sidxs[CC_NB]], sem_s,
                                add=True))
    for sd in sds:
        sd.wait()

    plsc.subcore_barrier()

    ods = [pltpu.async_copy(acc_cc.at[pl.ds(k * PLANE + s * ZCH, ZCH)],
                            out_cc.at[pl.ds(k * (CC_N * 128) + c * PLANE
                                            + s * ZCH, ZCH)], sem_z)
           for k in range(NKP)]
    ods.append(pltpu.async_copy(acc_dd.at[pl.ds(s * DD_SL, DD_SL)],
                                out_dd.at[pl.ds(c * F_DD + s * DD_SL,
                                                DD_SL)], sem_z))
    for od in ods:
        od.wait()


def _sc_build(cc_e2, cc_tl, dd_e2):
    mesh = plsc.VectorSubcoreMesh(core_axis_name="c", subcore_axis_name="s")
    return pl.kernel(
        _sc_body,
        out_type=(
            jax.ShapeDtypeStruct((NCORE * F_CC,), _f32),
            jax.ShapeDtypeStruct((NCORE * F_DD,), _f32),
        ),
        mesh=mesh,
        scratch_types=[
            pltpu.VMEM_SHARED((F_CC,), _f32),
            pltpu.VMEM_SHARED((F_DD,), _f32),
            pltpu.VMEM((2, CC_CH), _i32),
            pltpu.VMEM((2 * CC_T32,), _i32),
            pltpu.VMEM((2, DD_CH), _i32),
            [pltpu.VMEM((128,), _i32) for _ in range(CC_NB + 1)],
            [pltpu.VMEM((128,), _f32) for _ in range(CC_NB + 1)],
            pltpu.VMEM((ZCH,), _f32),
            pltpu.SemaphoreType.DMA,
            pltpu.SemaphoreType.DMA,
            pltpu.SemaphoreType.DMA,
        ],
    )(cc_e2, cc_tl, dd_e2)


def _tc_body(bcc_ref, bdd_ref, ccm_ref, ddm_ref, xc_ref, xd_ref,
             wc1, bc1, wc2, bc2, wd1, bd1, wd2, bd2,
             out_s, out_c, out_d):
    def gcn_stack(bm_ks, x, w1, b1, w2, b2, n):
        ones = jnp.ones((n, 1), _f32)
        cn = (((0,), (0,)), ((), ()))
        deg = jnp.concatenate(
            [lax.dot_general(bk, ones, cn, preferred_element_type=_f32)
             for bk in bm_ks], axis=0) + 1.0
        dinv = lax.rsqrt(deg)

        def layer(h, w, b):
            g = dinv * jnp.dot(h, w, preferred_element_type=_f32)
            m = jnp.concatenate(
                [lax.dot_general(bk, g, cn, preferred_element_type=_f32)
                 for bk in bm_ks], axis=0) + g
            return jnp.maximum(dinv * m + b, 0.0)

        h1 = layer(x, w1, b1)
        h2 = layer(h1, w2, b2)
        return jnp.concatenate([h1, h2], axis=1)

    bm_cc = []
    for k in range(NKP):
        w = min(128, NCIR - 128 * k)
        bm_cc.append(bcc_ref[k, 0:NCIR, 0:w]
                     * ccm_ref[0:NCIR, pl.ds(128 * k, w)])
    bm_dd = [bdd_ref[0:NDIS, 0:NDIS] * ddm_ref[...]]
    cir = gcn_stack(bm_cc, xc_ref[...], wc1[...], bc1[...], wc2[...],
                    bc2[...], NCIR)
    dis = gcn_stack(bm_dd, xd_ref[...], wd1[...], bd1[...], wd2[...],
                    bd2[...], NDIS)
    dis_p = jnp.concatenate(
        [dis, jnp.zeros((DD_N - NDIS, 2 * D), _f32)], axis=0)
    out_s[...] = lax.dot_general(cir, dis_p, (((1,), (1,)), ((), ())),
                                 preferred_element_type=_f32)
    out_c[...] = cir
    out_d[...] = dis


def _tc_dense(bcc, bdd, ccm, ddm, xc, xd,
              wc1, bc1, wc2, bc2, wd1, bd1, wd2, bd2):
    return pl.pallas_call(
        _tc_body,
        out_shape=(
            jax.ShapeDtypeStruct((NCIR, DD_N), _f32),
            jax.ShapeDtypeStruct((NCIR, 2 * D), _f32),
            jax.ShapeDtypeStruct((NDIS, 2 * D), _f32),
        ),
    )(bcc, bdd, ccm, ddm, xc, xd, wc1, bc1, wc2, bc2, wd1, bd1, wd2, bd2)


def kernel(cc_matrix, cc_edges, dd_matrix, dd_edges, x_cir, x_dis,
           W_cir1, b_cir1, W_cir2, b_cir2, W_dis1, b_dis1, W_dis2, b_dis2):
    out_cc, out_dd = _sc_build(
        cc_edges, cc_edges[:, CC_BLK:].reshape(2 * CC_T32), dd_edges)
    bcc = out_cc.reshape(NKP, CC_N, 128)
    bdd = out_dd.reshape(DD_N, DD_N)
    scores, cir, dis = _tc_dense(
        bcc, bdd, cc_matrix, dd_matrix, x_cir, x_dis,
        W_cir1, b_cir1.reshape(1, D), W_cir2, b_cir2.reshape(1, D),
        W_dis1, b_dis1.reshape(1, D), W_dis2, b_dis2.reshape(1, D))
    return (scores[:, :NDIS], cir, dis)
